# Initial kernel scaffold; baseline (speedup 1.0000x reference)
import jax
import jax.numpy as jnp
from jax.experimental import pallas as pl


def kernel(x, edge_index, edge_weight, W1, b1, W2, b2, W3, b3):
    raise NotImplementedError("write your pallas kernel here")


# speedup vs baseline: 5.8948x; 5.8948x over previous
import functools

import jax
import jax.numpy as jnp
from jax import lax
from jax.experimental import pallas as pl
from jax.experimental.pallas import tpu as pltpu
from jax.experimental.pallas import tpu_sc as plsc

N = 10000
NP = 10240
E = 320000
D = 128
NC = 2
NS = 16
NW = NC * NS
EPW = E // NW
CH = 128
NF = EPW // CH
TAIL = EPW - NF * CH
RPT = N // NS
PPT = NP // NS

_MESH = plsc.VectorSubcoreMesh(core_axis_name="c", subcore_axis_name="s")


def _worker_id():
    cid = lax.axis_index("c")
    sid = lax.axis_index("s")
    return cid, sid, sid * NC + cid


@functools.partial(
    pl.kernel,
    out_type=jax.ShapeDtypeStruct((NC * NP,), jnp.float32),
    mesh=_MESH,
    scratch_types=[
        pltpu.VMEM((CH,), jnp.int32),
        pltpu.VMEM((CH,), jnp.float32),
        pltpu.VMEM((TAIL,), jnp.int32),
        pltpu.VMEM((TAIL,), jnp.float32),
        pltpu.VMEM_SHARED((NP,), jnp.float32),
    ],
)
def _deg_kernel(row_hbm, w_hbm, zeros_hbm, out_hbm,
                idx_v, w_v, idx_t, w_t, acc_sh):
    cid, sid, wid = _worker_id()
    base = wid * EPW
    pltpu.sync_copy(zeros_hbm.at[pl.ds(sid * PPT, PPT)],
                    acc_sh.at[pl.ds(sid * PPT, PPT)])
    plsc.subcore_barrier()

    @pl.loop(0, NF)
    def _chunks(c):
        off = pl.multiple_of(base + c * CH, 8)
        pltpu.sync_copy(row_hbm.at[pl.ds(off, CH)], idx_v)
        pltpu.sync_copy(w_hbm.at[pl.ds(off, CH)], w_v)
        pltpu.sync_copy(w_v, acc_sh.at[idx_v], add=True)

    offt = pl.multiple_of(base + NF * CH, 8)
    pltpu.sync_copy(row_hbm.at[pl.ds(offt, TAIL)], idx_t)
    pltpu.sync_copy(w_hbm.at[pl.ds(offt, TAIL)], w_t)
    pltpu.sync_copy(w_t, acc_sh.at[idx_t], add=True)

    plsc.subcore_barrier()
    pltpu.sync_copy(acc_sh.at[pl.ds(sid * PPT, PPT)],
                    out_hbm.at[pl.ds(cid * NP + sid * PPT, PPT)])


@functools.partial(
    pl.kernel,
    out_type=jax.ShapeDtypeStruct((E,), jnp.float32),
    mesh=_MESH,
    scratch_types=[
        pltpu.VMEM((CH,), jnp.int32),
        pltpu.VMEM((CH,), jnp.int32),
        pltpu.VMEM((CH,), jnp.float32),
        pltpu.VMEM((CH,), jnp.float32),
        pltpu.VMEM((CH,), jnp.float32),
        pltpu.VMEM((CH,), jnp.float32),
        pltpu.SemaphoreType.DMA,
        pltpu.SemaphoreType.DMA,
    ],
)
def _norm_kernel(row_hbm, col_hbm, w_hbm, dinv_hbm, out_hbm,
                 ridx_v, cidx_v, w_v, dr_v, dc_v, nrm_v, sem_r, sem_c):
    _, _, wid = _worker_id()
    base = wid * EPW

    def do_chunk(off, n, ridx, cidx, wv, dr, dc, nrm):
        pltpu.sync_copy(row_hbm.at[pl.ds(off, n)], ridx)
        pltpu.sync_copy(col_hbm.at[pl.ds(off, n)], cidx)
        pltpu.sync_copy(w_hbm.at[pl.ds(off, n)], wv)
        a = pltpu.async_copy(dinv_hbm.at[ridx], dr, sem_r)
        b = pltpu.async_copy(dinv_hbm.at[cidx], dc, sem_c)
        a.wait()
        b.wait()
        for j in range(n // 16):
            sl = pl.ds(j * 16, 16)
            nrm[sl] = -(dr[sl] * wv[sl] * dc[sl])
        pltpu.sync_copy(nrm, out_hbm.at[pl.ds(off, n)])

    @pl.loop(0, NF)
    def _chunks(c):
        do_chunk(pl.multiple_of(base + c * CH, 8), CH,
                 ridx_v, cidx_v, w_v, dr_v, dc_v, nrm_v)

    offt = pl.multiple_of(base + NF * CH, 8)
    pltpu.sync_copy(row_hbm.at[pl.ds(offt, TAIL)], ridx_v.at[pl.ds(0, TAIL)])
    pltpu.sync_copy(col_hbm.at[pl.ds(offt, TAIL)], cidx_v.at[pl.ds(0, TAIL)])
    pltpu.sync_copy(w_hbm.at[pl.ds(offt, TAIL)], w_v.at[pl.ds(0, TAIL)])
    a = pltpu.async_copy(dinv_hbm.at[ridx_v.at[pl.ds(0, TAIL)]],
                         dr_v.at[pl.ds(0, TAIL)], sem_r)
    a.wait()
    b = pltpu.async_copy(dinv_hbm.at[cidx_v.at[pl.ds(0, TAIL)]],
                         dc_v.at[pl.ds(0, TAIL)], sem_c)
    b.wait()
    sl = pl.ds(0, 16)
    nrm_t = -(dr_v[sl] * w_v[sl] * dc_v[sl])
    nrm_v[sl] = nrm_t
    pltpu.sync_copy(nrm_v.at[pl.ds(0, TAIL)], out_hbm.at[pl.ds(offt, TAIL)])


@functools.partial(
    pl.kernel,
    out_type=jax.ShapeDtypeStruct((NC, NP, D), jnp.float32),
    mesh=_MESH,
    scratch_types=[
        pltpu.VMEM((CH,), jnp.int32),
        pltpu.VMEM((CH,), jnp.int32),
        pltpu.VMEM((CH,), jnp.float32),
        pltpu.VMEM((CH, D), jnp.float32),
        pltpu.VMEM((TAIL,), jnp.int32),
        pltpu.VMEM((TAIL,), jnp.int32),
        pltpu.VMEM((TAIL,), jnp.float32),
        pltpu.VMEM((TAIL, D), jnp.float32),
        pltpu.VMEM_SHARED((NP, D), jnp.float32),
        pltpu.SemaphoreType.DMA,
    ],
)
def _spmm_kernel(x_hbm, row_hbm, col_hbm, nrm_hbm, zeros_hbm, out_hbm,
                 ridx_v, cidx_v, nrm_v, rows_v,
                 ridx_t, cidx_t, nrm_t, rows_t, acc_sh, sem):
    cid, sid, wid = _worker_id()
    base = wid * EPW
    pltpu.sync_copy(zeros_hbm.at[pl.ds(sid * PPT, PPT)],
                    acc_sh.at[pl.ds(sid * PPT, PPT)])
    plsc.subcore_barrier()

    def scale_rows(rows, nrm, n):
        for g in range(n // 16):
            nv = nrm[pl.ds(g * 16, 16)]
            for j in range(16):
                e = g * 16 + j
                spl = lax.gather(
                    nv, jnp.full((16, 1), j, jnp.int32),
                    lax.GatherDimensionNumbers(
                        offset_dims=(), collapsed_slice_dims=(0,),
                        start_index_map=(0,)),
                    slice_sizes=(1,),
                    mode=lax.GatherScatterMode.PROMISE_IN_BOUNDS)
                for s in range(D // 16):
                    sl = pl.ds(s * 16, 16)
                    rows[e, sl] = rows[e, sl] * spl

    @pl.loop(0, NF)
    def _chunks(c):
        off = pl.multiple_of(base + c * CH, 8)
        pltpu.sync_copy(row_hbm.at[pl.ds(off, CH)], ridx_v)
        pltpu.sync_copy(col_hbm.at[pl.ds(off, CH)], cidx_v)
        pltpu.sync_copy(nrm_hbm.at[pl.ds(off, CH)], nrm_v)
        pltpu.async_copy(x_hbm.at[ridx_v], rows_v, sem).wait()
        scale_rows(rows_v, nrm_v, CH)
        pltpu.sync_copy(rows_v, acc_sh.at[cidx_v], add=True)

    offt = pl.multiple_of(base + NF * CH, 8)
    pltpu.sync_copy(row_hbm.at[pl.ds(offt, TAIL)], ridx_t)
    pltpu.sync_copy(col_hbm.at[pl.ds(offt, TAIL)], cidx_t)
    pltpu.sync_copy(nrm_hbm.at[pl.ds(offt, TAIL)], nrm_t)
    pltpu.async_copy(x_hbm.at[ridx_t], rows_t, sem).wait()
    scale_rows(rows_t, nrm_t, TAIL)
    pltpu.sync_copy(rows_t, acc_sh.at[cidx_t], add=True)

    plsc.subcore_barrier()
    pltpu.sync_copy(acc_sh.at[pl.ds(sid * PPT, PPT)],
                    out_hbm.at[cid, pl.ds(sid * PPT, PPT)])


def _dinv_body(degp_ref, out_ref):
    s = degp_ref[0] + degp_ref[1]
    out_ref[...] = jnp.where(s > 0, lax.rsqrt(s), 0.0)


def _combine_body(p_ref, out_ref):
    out_ref[...] = p_ref[0] + p_ref[1]


def _layer_body(h_ref, s1_ref, p2_ref, w0_ref, w1_ref, w2_ref, b_ref, out_ref):
    w0 = w0_ref[...] - w2_ref[...]
    w2 = 2.0 * w2_ref[...]
    t2 = p2_ref[0] + p2_ref[1]
    acc = jnp.dot(h_ref[...], w0, preferred_element_type=jnp.float32)
    acc += jnp.dot(s1_ref[...], w1_ref[...], preferred_element_type=jnp.float32)
    acc += jnp.dot(t2, w2, preferred_element_type=jnp.float32)
    acc += b_ref[...]
    out_ref[...] = 1.0 / (1.0 + jnp.exp(-acc))


_RB = 1024

_combine = pl.pallas_call(
    _combine_body,
    grid=(NP // _RB,),
    in_specs=[pl.BlockSpec((NC, _RB, D), lambda i: (0, i, 0))],
    out_specs=pl.BlockSpec((_RB, D), lambda i: (i, 0)),
    out_shape=jax.ShapeDtypeStruct((NP, D), jnp.float32),
)

_layer = pl.pallas_call(
    _layer_body,
    grid=(NP // _RB,),
    in_specs=[
        pl.BlockSpec((_RB, D), lambda i: (i, 0)),
        pl.BlockSpec((_RB, D), lambda i: (i, 0)),
        pl.BlockSpec((NC, _RB, D), lambda i: (0, i, 0)),
        pl.BlockSpec((D, D), lambda i: (0, 0)),
        pl.BlockSpec((D, D), lambda i: (0, 0)),
        pl.BlockSpec((D, D), lambda i: (0, 0)),
        pl.BlockSpec((1, D), lambda i: (0, 0)),
    ],
    out_specs=pl.BlockSpec((_RB, D), lambda i: (i, 0)),
    out_shape=jax.ShapeDtypeStruct((NP, D), jnp.float32),
)

_dinv = pl.pallas_call(
    _dinv_body,
    out_shape=jax.ShapeDtypeStruct((NP // D, D), jnp.float32),
)


def kernel(x, edge_index, edge_weight, W1, b1, W2, b2, W3, b3):
    row = edge_index[0]
    col = edge_index[1]
    zeros_np = jnp.zeros((NP,), jnp.float32)
    zeros_nd = jnp.zeros((NP, D), jnp.float32)

    degp = _deg_kernel(row, edge_weight, zeros_np)
    dinv = _dinv(degp.reshape(NC, NP // D, D)).reshape(NP)
    norm = _norm_kernel(row, col, edge_weight, dinv)

    h = jnp.pad(x, ((0, NP - N), (0, 0)))
    for W, b in ((W1, b1), (W2, b2), (W3, b3)):
        p1 = _spmm_kernel(h, row, col, norm, zeros_nd)
        s1 = _combine(p1)
        p2 = _spmm_kernel(s1, row, col, norm, zeros_nd)
        h = _layer(h, s1, p2, W[0], W[1], W[2], b.reshape(1, D))
    return h[:N]


# speedup vs baseline: 6.8383x; 1.1601x over previous
import functools

import jax
import jax.numpy as jnp
from jax import lax
from jax.experimental import pallas as pl
from jax.experimental.pallas import tpu as pltpu
from jax.experimental.pallas import tpu_sc as plsc

N = 10000
NP = 10240
E = 320000
D = 128
NC = 2
NS = 16
NW = NC * NS
EPW = E // NW
CH = 128
NF = EPW // CH
TAIL = EPW - NF * CH
RPT = N // NS
PPT = NP // NS

_MESH = plsc.VectorSubcoreMesh(core_axis_name="c", subcore_axis_name="s")


def _worker_id():
    cid = lax.axis_index("c")
    sid = lax.axis_index("s")
    return cid, sid, sid * NC + cid


@functools.partial(
    pl.kernel,
    out_type=jax.ShapeDtypeStruct((NC * NP,), jnp.float32),
    mesh=_MESH,
    scratch_types=[
        pltpu.VMEM((CH,), jnp.int32),
        pltpu.VMEM((CH,), jnp.float32),
        pltpu.VMEM((TAIL,), jnp.int32),
        pltpu.VMEM((TAIL,), jnp.float32),
        pltpu.VMEM_SHARED((NP,), jnp.float32),
    ],
)
def _deg_kernel(row_hbm, w_hbm, zeros_hbm, out_hbm,
                idx_v, w_v, idx_t, w_t, acc_sh):
    cid, sid, wid = _worker_id()
    base = wid * EPW
    pltpu.sync_copy(zeros_hbm.at[pl.ds(sid * PPT, PPT)],
                    acc_sh.at[pl.ds(sid * PPT, PPT)])
    plsc.subcore_barrier()

    @pl.loop(0, NF)
    def _chunks(c):
        off = pl.multiple_of(base + c * CH, 8)
        pltpu.sync_copy(row_hbm.at[pl.ds(off, CH)], idx_v)
        pltpu.sync_copy(w_hbm.at[pl.ds(off, CH)], w_v)
        pltpu.sync_copy(w_v, acc_sh.at[idx_v], add=True)

    offt = pl.multiple_of(base + NF * CH, 8)
    pltpu.sync_copy(row_hbm.at[pl.ds(offt, TAIL)], idx_t)
    pltpu.sync_copy(w_hbm.at[pl.ds(offt, TAIL)], w_t)
    pltpu.sync_copy(w_t, acc_sh.at[idx_t], add=True)

    plsc.subcore_barrier()
    pltpu.sync_copy(acc_sh.at[pl.ds(sid * PPT, PPT)],
                    out_hbm.at[pl.ds(cid * NP + sid * PPT, PPT)])


NCHUNK = E // CH
NFULLR = NCHUNK // NW
NEXTRA = NCHUNK - NFULLR * NW


@functools.partial(
    pl.kernel,
    out_type=jax.ShapeDtypeStruct((NCHUNK, 3, CH), jnp.int32),
    mesh=_MESH,
    scratch_types=[
        pltpu.VMEM((3, CH), jnp.int32),
        pltpu.VMEM((CH,), jnp.float32),
        pltpu.VMEM((CH,), jnp.float32),
        pltpu.VMEM((CH,), jnp.float32),
        pltpu.SemaphoreType.DMA,
        pltpu.SemaphoreType.DMA,
    ],
)
def _norm_kernel(row_hbm, col_hbm, w_hbm, dinv_hbm, out_hbm,
                 ebuf, w_v, dr_v, dc_v, sem_r, sem_c):
    _, _, wid = _worker_id()

    def do_chunk(c):
        off = pl.multiple_of(c * CH, 8)
        pltpu.sync_copy(row_hbm.at[pl.ds(off, CH)], ebuf.at[0])
        pltpu.sync_copy(col_hbm.at[pl.ds(off, CH)], ebuf.at[1])
        pltpu.sync_copy(w_hbm.at[pl.ds(off, CH)], w_v)
        a = pltpu.async_copy(dinv_hbm.at[ebuf.at[0]], dr_v, sem_r)
        b = pltpu.async_copy(dinv_hbm.at[ebuf.at[1]], dc_v, sem_c)
        a.wait()
        b.wait()
        for j in range(CH // 16):
            sl = pl.ds(j * 16, 16)
            ebuf[2, sl] = lax.bitcast_convert_type(
                -(dr_v[sl] * w_v[sl] * dc_v[sl]), jnp.int32)
        pltpu.sync_copy(ebuf, out_hbm.at[c])

    @pl.loop(0, NFULLR)
    def _chunks(g):
        do_chunk(g * NW + wid)

    @pl.when(wid < NEXTRA)
    def _extra():
        do_chunk(NFULLR * NW + wid)


NB = 2
NOUT = NFULLR // NB


@functools.partial(
    pl.kernel,
    out_type=jax.ShapeDtypeStruct((NC, NP, D), jnp.float32),
    mesh=_MESH,
    scratch_types=[
        [pltpu.VMEM((3, CH), jnp.int32) for _ in range(NB)],
        [pltpu.VMEM((CH, D), jnp.float32) for _ in range(NB)],
        pltpu.VMEM_SHARED((NP, D), jnp.float32),
        [pltpu.SemaphoreType.DMA for _ in range(NB)],
        [pltpu.SemaphoreType.DMA for _ in range(NB)],
        [pltpu.SemaphoreType.DMA for _ in range(NB)],
    ],
)
def _spmm_kernel(x_hbm, edata_hbm, zeros_hbm, out_hbm,
                 ebufs, rowss, acc_sh, sems_e, sems_g, sems_s):
    cid, sid, wid = _worker_id()
    pltpu.sync_copy(zeros_hbm.at[pl.ds(sid * PPT, PPT)],
                    acc_sh.at[pl.ds(sid * PPT, PPT)])
    plsc.subcore_barrier()

    def scale_rows(rows, ebuf):
        for g in range(CH // 16):
            nv = lax.bitcast_convert_type(ebuf[2, pl.ds(g * 16, 16)], jnp.float32)
            for j in range(16):
                e = g * 16 + j
                spl = lax.gather(
                    nv, jnp.full((16, 1), j, jnp.int32),
                    lax.GatherDimensionNumbers(
                        offset_dims=(), collapsed_slice_dims=(0,),
                        start_index_map=(0,)),
                    slice_sizes=(1,),
                    mode=lax.GatherScatterMode.PROMISE_IN_BOUNDS)
                for s in range(D // 16):
                    sl = pl.ds(s * 16, 16)
                    rows[e, sl] = rows[e, sl] * spl

    @pl.loop(0, NOUT)
    def _outer(g):
        first = g * NB * NW + wid
        de = [pltpu.async_copy(edata_hbm.at[first + b * NW], ebufs[b],
                               sems_e[b]) for b in range(NB)]
        dg = []
        for b in range(NB):
            de[b].wait()
            dg.append(pltpu.async_copy(x_hbm.at[ebufs[b].at[0]], rowss[b],
                                       sems_g[b]))
        for b in range(NB):
            dg[b].wait()
            scale_rows(rowss[b], ebufs[b])
            pltpu.sync_copy(rowss[b], acc_sh.at[ebufs[b].at[1]], add=True)

    @pl.when(wid < NEXTRA)
    def _extra():
        c = NFULLR * NW + wid
        pltpu.async_copy(edata_hbm.at[c], ebufs[0], sems_e[0]).wait()
        pltpu.async_copy(x_hbm.at[ebufs[0].at[0]], rowss[0], sems_g[0]).wait()
        scale_rows(rowss[0], ebufs[0])
        pltpu.sync_copy(rowss[0], acc_sh.at[ebufs[0].at[1]], add=True)

    plsc.subcore_barrier()
    pltpu.sync_copy(acc_sh.at[pl.ds(sid * PPT, PPT)],
                    out_hbm.at[cid, pl.ds(sid * PPT, PPT)])


def _dinv_body(degp_ref, out_ref):
    s = degp_ref[0] + degp_ref[1]
    out_ref[...] = jnp.where(s > 0, lax.rsqrt(s), 0.0)


def _combine_body(p_ref, out_ref):
    out_ref[...] = p_ref[0] + p_ref[1]


def _layer_body(h_ref, s1_ref, p2_ref, w0_ref, w1_ref, w2_ref, b_ref, out_ref):
    w0 = w0_ref[...] - w2_ref[...]
    w2 = 2.0 * w2_ref[...]
    t2 = p2_ref[0] + p2_ref[1]
    acc = jnp.dot(h_ref[...], w0, preferred_element_type=jnp.float32)
    acc += jnp.dot(s1_ref[...], w1_ref[...], preferred_element_type=jnp.float32)
    acc += jnp.dot(t2, w2, preferred_element_type=jnp.float32)
    acc += b_ref[...]
    out_ref[...] = 1.0 / (1.0 + jnp.exp(-acc))


_RB = 1024

_combine = pl.pallas_call(
    _combine_body,
    grid=(NP // _RB,),
    in_specs=[pl.BlockSpec((NC, _RB, D), lambda i: (0, i, 0))],
    out_specs=pl.BlockSpec((_RB, D), lambda i: (i, 0)),
    out_shape=jax.ShapeDtypeStruct((NP, D), jnp.float32),
)

_layer = pl.pallas_call(
    _layer_body,
    grid=(NP // _RB,),
    in_specs=[
        pl.BlockSpec((_RB, D), lambda i: (i, 0)),
        pl.BlockSpec((_RB, D), lambda i: (i, 0)),
        pl.BlockSpec((NC, _RB, D), lambda i: (0, i, 0)),
        pl.BlockSpec((D, D), lambda i: (0, 0)),
        pl.BlockSpec((D, D), lambda i: (0, 0)),
        pl.BlockSpec((D, D), lambda i: (0, 0)),
        pl.BlockSpec((1, D), lambda i: (0, 0)),
    ],
    out_specs=pl.BlockSpec((_RB, D), lambda i: (i, 0)),
    out_shape=jax.ShapeDtypeStruct((NP, D), jnp.float32),
)

_dinv = pl.pallas_call(
    _dinv_body,
    out_shape=jax.ShapeDtypeStruct((NP // D, D), jnp.float32),
)


def kernel(x, edge_index, edge_weight, W1, b1, W2, b2, W3, b3):
    row = edge_index[0]
    col = edge_index[1]
    zeros_np = jnp.zeros((NP,), jnp.float32)
    zeros_nd = jnp.zeros((NP, D), jnp.float32)

    degp = _deg_kernel(row, edge_weight, zeros_np)
    dinv = _dinv(degp.reshape(NC, NP // D, D)).reshape(NP)
    edata = _norm_kernel(row, col, edge_weight, dinv)

    h = jnp.pad(x, ((0, NP - N), (0, 0)))
    for W, b in ((W1, b1), (W2, b2), (W3, b3)):
        p1 = _spmm_kernel(h, edata, zeros_nd)
        s1 = _combine(p1)
        p2 = _spmm_kernel(s1, edata, zeros_nd)
        h = _layer(h, s1, p2, W[0], W[1], W[2], b.reshape(1, D))
    return h[:N]


# speedup vs baseline: 7.7564x; 1.1343x over previous
import functools

import jax
import jax.numpy as jnp
from jax import lax
from jax.experimental import pallas as pl
from jax.experimental.pallas import tpu as pltpu
from jax.experimental.pallas import tpu_sc as plsc

N = 10000
NP = 10240
E = 320000
D = 128
NC = 2
NS = 16
NW = NC * NS
EPW = E // NW
CH = 128
NF = EPW // CH
TAIL = EPW - NF * CH
RPT = N // NS
PPT = NP // NS

_MESH = plsc.VectorSubcoreMesh(core_axis_name="c", subcore_axis_name="s")


def _worker_id():
    cid = lax.axis_index("c")
    sid = lax.axis_index("s")
    return cid, sid, sid * NC + cid


@functools.partial(
    pl.kernel,
    out_type=jax.ShapeDtypeStruct((NC * NP,), jnp.float32),
    mesh=_MESH,
    scratch_types=[
        pltpu.VMEM((CH,), jnp.int32),
        pltpu.VMEM((CH,), jnp.float32),
        pltpu.VMEM((TAIL,), jnp.int32),
        pltpu.VMEM((TAIL,), jnp.float32),
        pltpu.VMEM_SHARED((NP,), jnp.float32),
    ],
)
def _deg_kernel(row_hbm, w_hbm, zeros_hbm, out_hbm,
                idx_v, w_v, idx_t, w_t, acc_sh):
    cid, sid, wid = _worker_id()
    base = wid * EPW
    pltpu.sync_copy(zeros_hbm.at[pl.ds(sid * PPT, PPT)],
                    acc_sh.at[pl.ds(sid * PPT, PPT)])
    plsc.subcore_barrier()

    @pl.loop(0, NF)
    def _chunks(c):
        off = pl.multiple_of(base + c * CH, 8)
        pltpu.sync_copy(row_hbm.at[pl.ds(off, CH)], idx_v)
        pltpu.sync_copy(w_hbm.at[pl.ds(off, CH)], w_v)
        pltpu.sync_copy(w_v, acc_sh.at[idx_v], add=True)

    offt = pl.multiple_of(base + NF * CH, 8)
    pltpu.sync_copy(row_hbm.at[pl.ds(offt, TAIL)], idx_t)
    pltpu.sync_copy(w_hbm.at[pl.ds(offt, TAIL)], w_t)
    pltpu.sync_copy(w_t, acc_sh.at[idx_t], add=True)

    plsc.subcore_barrier()
    pltpu.sync_copy(acc_sh.at[pl.ds(sid * PPT, PPT)],
                    out_hbm.at[pl.ds(cid * NP + sid * PPT, PPT)])


NCHUNK = E // CH
NFULLR = NCHUNK // NW
NEXTRA = NCHUNK - NFULLR * NW


@functools.partial(
    pl.kernel,
    out_type=jax.ShapeDtypeStruct((NCHUNK, 3, CH), jnp.int32),
    mesh=_MESH,
    scratch_types=[
        pltpu.VMEM((3, CH), jnp.int32),
        pltpu.VMEM((CH,), jnp.float32),
        pltpu.VMEM((CH,), jnp.float32),
        pltpu.VMEM((CH,), jnp.float32),
        pltpu.SemaphoreType.DMA,
        pltpu.SemaphoreType.DMA,
    ],
)
def _norm_kernel(row_hbm, col_hbm, w_hbm, dinv_hbm, out_hbm,
                 ebuf, w_v, dr_v, dc_v, sem_r, sem_c):
    _, _, wid = _worker_id()

    def do_chunk(c):
        off = pl.multiple_of(c * CH, 8)
        pltpu.sync_copy(row_hbm.at[pl.ds(off, CH)], ebuf.at[0])
        pltpu.sync_copy(col_hbm.at[pl.ds(off, CH)], ebuf.at[1])
        pltpu.sync_copy(w_hbm.at[pl.ds(off, CH)], w_v)
        a = pltpu.async_copy(dinv_hbm.at[ebuf.at[0]], dr_v, sem_r)
        b = pltpu.async_copy(dinv_hbm.at[ebuf.at[1]], dc_v, sem_c)
        a.wait()
        b.wait()
        for j in range(CH // 16):
            sl = pl.ds(j * 16, 16)
            ebuf[2, sl] = lax.bitcast_convert_type(
                -(dr_v[sl] * w_v[sl] * dc_v[sl]), jnp.int32)
        pltpu.sync_copy(ebuf, out_hbm.at[c])

    @pl.loop(0, NFULLR)
    def _chunks(g):
        do_chunk(g * NW + wid)

    @pl.when(wid < NEXTRA)
    def _extra():
        do_chunk(NFULLR * NW + wid)


NB = 2
NOUT = NFULLR // NB


@functools.partial(
    pl.kernel,
    out_type=jax.ShapeDtypeStruct((NC, NP, D), jnp.float32),
    mesh=_MESH,
    scratch_types=[
        [pltpu.VMEM((3, CH), jnp.int32) for _ in range(NB)],
        [pltpu.VMEM((CH, D), jnp.float32) for _ in range(NB)],
        pltpu.VMEM_SHARED((NP, D), jnp.float32),
        [pltpu.SemaphoreType.DMA for _ in range(NB)],
        [pltpu.SemaphoreType.DMA for _ in range(NB)],
        [pltpu.SemaphoreType.DMA for _ in range(NB)],
    ],
)
def _spmm_kernel(x_hbm, edata_hbm, zeros_hbm, out_hbm,
                 ebufs, rowss, acc_sh, sems_e, sems_g, sems_s):
    cid, sid, wid = _worker_id()
    pltpu.sync_copy(zeros_hbm.at[pl.ds(sid * PPT, PPT)],
                    acc_sh.at[pl.ds(sid * PPT, PPT)])
    plsc.subcore_barrier()

    def scale_rows(rows, ebuf):
        for g in range(CH // 16):
            nv = lax.bitcast_convert_type(ebuf[2, pl.ds(g * 16, 16)], jnp.float32)
            for j in range(16):
                e = g * 16 + j
                spl = lax.gather(
                    nv, jnp.full((16, 1), j, jnp.int32),
                    lax.GatherDimensionNumbers(
                        offset_dims=(), collapsed_slice_dims=(0,),
                        start_index_map=(0,)),
                    slice_sizes=(1,),
                    mode=lax.GatherScatterMode.PROMISE_IN_BOUNDS)
                for s in range(D // 16):
                    sl = pl.ds(s * 16, 16)
                    rows[e, sl] = rows[e, sl] * spl

    for b in range(NB):
        pltpu.async_copy(edata_hbm.at[b * NW + wid], ebufs[b], sems_e[b])

    @pl.loop(0, NOUT)
    def _outer(g):
        dg = []
        for b in range(NB):
            pltpu.make_async_copy(edata_hbm.at[0], ebufs[b], sems_e[b]).wait()
            dg.append(pltpu.async_copy(x_hbm.at[ebufs[b].at[0]], rowss[b],
                                       sems_g[b]))
        ds = []
        for b in range(NB):
            dg[b].wait()
            scale_rows(rowss[b], ebufs[b])
            ds.append(pltpu.async_copy(rowss[b], acc_sh.at[ebufs[b].at[1]],
                                       sems_s[b], add=True))
        for b in range(NB):
            ds[b].wait()

            @pl.when(g < NOUT - 1)
            def _prefetch():
                pltpu.async_copy(
                    edata_hbm.at[((g + 1) * NB + b) * NW + wid],
                    ebufs[b], sems_e[b])

    @pl.when(wid < NEXTRA)
    def _extra():
        c = NFULLR * NW + wid
        pltpu.async_copy(edata_hbm.at[c], ebufs[0], sems_e[0]).wait()
        pltpu.async_copy(x_hbm.at[ebufs[0].at[0]], rowss[0], sems_g[0]).wait()
        scale_rows(rowss[0], ebufs[0])
        pltpu.async_copy(rowss[0], acc_sh.at[ebufs[0].at[1]],
                         sems_s[0], add=True).wait()

    plsc.subcore_barrier()
    pltpu.sync_copy(acc_sh.at[pl.ds(sid * PPT, PPT)],
                    out_hbm.at[cid, pl.ds(sid * PPT, PPT)])


def _dinv_body(degp_ref, out_ref):
    s = degp_ref[0] + degp_ref[1]
    out_ref[...] = jnp.where(s > 0, lax.rsqrt(s), 0.0)


def _combine_body(p_ref, out_ref):
    out_ref[...] = p_ref[0] + p_ref[1]


def _layer_body(h_ref, s1_ref, p2_ref, w0_ref, w1_ref, w2_ref, b_ref, out_ref):
    w0 = w0_ref[...] - w2_ref[...]
    w2 = 2.0 * w2_ref[...]
    t2 = p2_ref[0] + p2_ref[1]
    acc = jnp.dot(h_ref[...], w0, preferred_element_type=jnp.float32)
    acc += jnp.dot(s1_ref[...], w1_ref[...], preferred_element_type=jnp.float32)
    acc += jnp.dot(t2, w2, preferred_element_type=jnp.float32)
    acc += b_ref[...]
    out_ref[...] = 1.0 / (1.0 + jnp.exp(-acc))


_RB = 1024

_combine = pl.pallas_call(
    _combine_body,
    grid=(NP // _RB,),
    in_specs=[pl.BlockSpec((NC, _RB, D), lambda i: (0, i, 0))],
    out_specs=pl.BlockSpec((_RB, D), lambda i: (i, 0)),
    out_shape=jax.ShapeDtypeStruct((NP, D), jnp.float32),
)

_layer = pl.pallas_call(
    _layer_body,
    grid=(NP // _RB,),
    in_specs=[
        pl.BlockSpec((_RB, D), lambda i: (i, 0)),
        pl.BlockSpec((_RB, D), lambda i: (i, 0)),
        pl.BlockSpec((NC, _RB, D), lambda i: (0, i, 0)),
        pl.BlockSpec((D, D), lambda i: (0, 0)),
        pl.BlockSpec((D, D), lambda i: (0, 0)),
        pl.BlockSpec((D, D), lambda i: (0, 0)),
        pl.BlockSpec((1, D), lambda i: (0, 0)),
    ],
    out_specs=pl.BlockSpec((_RB, D), lambda i: (i, 0)),
    out_shape=jax.ShapeDtypeStruct((NP, D), jnp.float32),
)

_dinv = pl.pallas_call(
    _dinv_body,
    out_shape=jax.ShapeDtypeStruct((NP // D, D), jnp.float32),
)


def kernel(x, edge_index, edge_weight, W1, b1, W2, b2, W3, b3):
    row = edge_index[0]
    col = edge_index[1]
    zeros_np = jnp.zeros((NP,), jnp.float32)
    zeros_nd = jnp.zeros((NP, D), jnp.float32)

    degp = _deg_kernel(row, edge_weight, zeros_np)
    dinv = _dinv(degp.reshape(NC, NP // D, D)).reshape(NP)
    edata = _norm_kernel(row, col, edge_weight, dinv)

    h = jnp.pad(x, ((0, NP - N), (0, 0)))
    for W, b in ((W1, b1), (W2, b2), (W3, b3)):
        p1 = _spmm_kernel(h, edata, zeros_nd)
        s1 = _combine(p1)
        p2 = _spmm_kernel(s1, edata, zeros_nd)
        h = _layer(h, s1, p2, W[0], W[1], W[2], b.reshape(1, D))
    return h[:N]


# speedup vs baseline: 8.0964x; 1.0438x over previous
import functools

import jax
import jax.numpy as jnp
from jax import lax
from jax.experimental import pallas as pl
from jax.experimental.pallas import tpu as pltpu
from jax.experimental.pallas import tpu_sc as plsc

N = 10000
NP = 10240
E = 320000
D = 128
NC = 2
NS = 16
NW = NC * NS
EPW = E // NW
CH = 128
NF = EPW // CH
TAIL = EPW - NF * CH
RPT = N // NS
PPT = NP // NS

_MESH = plsc.VectorSubcoreMesh(core_axis_name="c", subcore_axis_name="s")


def _worker_id():
    cid = lax.axis_index("c")
    sid = lax.axis_index("s")
    return cid, sid, sid * NC + cid


@functools.partial(
    pl.kernel,
    out_type=jax.ShapeDtypeStruct((NC * NP,), jnp.float32),
    mesh=_MESH,
    scratch_types=[
        pltpu.VMEM((CH,), jnp.int32),
        pltpu.VMEM((CH,), jnp.float32),
        pltpu.VMEM((TAIL,), jnp.int32),
        pltpu.VMEM((TAIL,), jnp.float32),
        pltpu.VMEM_SHARED((NP,), jnp.float32),
    ],
)
def _deg_kernel(row_hbm, w_hbm, zeros_hbm, out_hbm,
                idx_v, w_v, idx_t, w_t, acc_sh):
    cid, sid, wid = _worker_id()
    base = wid * EPW
    pltpu.sync_copy(zeros_hbm.at[pl.ds(sid * PPT, PPT)],
                    acc_sh.at[pl.ds(sid * PPT, PPT)])
    plsc.subcore_barrier()

    @pl.loop(0, NF)
    def _chunks(c):
        off = pl.multiple_of(base + c * CH, 8)
        pltpu.sync_copy(row_hbm.at[pl.ds(off, CH)], idx_v)
        pltpu.sync_copy(w_hbm.at[pl.ds(off, CH)], w_v)
        pltpu.sync_copy(w_v, acc_sh.at[idx_v], add=True)

    offt = pl.multiple_of(base + NF * CH, 8)
    pltpu.sync_copy(row_hbm.at[pl.ds(offt, TAIL)], idx_t)
    pltpu.sync_copy(w_hbm.at[pl.ds(offt, TAIL)], w_t)
    pltpu.sync_copy(w_t, acc_sh.at[idx_t], add=True)

    plsc.subcore_barrier()
    pltpu.sync_copy(acc_sh.at[pl.ds(sid * PPT, PPT)],
                    out_hbm.at[pl.ds(cid * NP + sid * PPT, PPT)])


NCHUNK = E // CH
NFULLR = NCHUNK // NW
NEXTRA = NCHUNK - NFULLR * NW


@functools.partial(
    pl.kernel,
    out_type=jax.ShapeDtypeStruct((NCHUNK, 3, CH), jnp.int32),
    mesh=_MESH,
    scratch_types=[
        pltpu.VMEM((3, CH), jnp.int32),
        pltpu.VMEM((CH,), jnp.float32),
        pltpu.VMEM((CH,), jnp.float32),
        pltpu.VMEM((CH,), jnp.float32),
        pltpu.SemaphoreType.DMA,
        pltpu.SemaphoreType.DMA,
    ],
)
def _norm_kernel(row_hbm, col_hbm, w_hbm, dinv_hbm, out_hbm,
                 ebuf, w_v, dr_v, dc_v, sem_r, sem_c):
    _, _, wid = _worker_id()

    def do_chunk(c):
        off = pl.multiple_of(c * CH, 8)
        pltpu.sync_copy(row_hbm.at[pl.ds(off, CH)], ebuf.at[0])
        pltpu.sync_copy(col_hbm.at[pl.ds(off, CH)], ebuf.at[1])
        pltpu.sync_copy(w_hbm.at[pl.ds(off, CH)], w_v)
        a = pltpu.async_copy(dinv_hbm.at[ebuf.at[0]], dr_v, sem_r)
        b = pltpu.async_copy(dinv_hbm.at[ebuf.at[1]], dc_v, sem_c)
        a.wait()
        b.wait()
        for j in range(CH // 16):
            sl = pl.ds(j * 16, 16)
            ebuf[2, sl] = lax.bitcast_convert_type(
                -(dr_v[sl] * w_v[sl] * dc_v[sl]), jnp.int32)
        pltpu.sync_copy(ebuf, out_hbm.at[c])

    @pl.loop(0, NFULLR)
    def _chunks(g):
        do_chunk(g * NW + wid)

    @pl.when(wid < NEXTRA)
    def _extra():
        do_chunk(NFULLR * NW + wid)


NB = 2
NOUT = NFULLR // NB


@functools.partial(
    pl.kernel,
    out_type=jax.ShapeDtypeStruct((NC, NP, D), jnp.float32),
    mesh=_MESH,
    scratch_types=[
        [pltpu.VMEM((3, CH), jnp.int32) for _ in range(NB)],
        [pltpu.VMEM((CH,), jnp.int32) for _ in range(NB)],
        [pltpu.VMEM((CH,), jnp.int32) for _ in range(NB)],
        [pltpu.VMEM((CH,), jnp.int32) for _ in range(NB)],
        [pltpu.VMEM((CH, D), jnp.float32) for _ in range(NB)],
        pltpu.VMEM_SHARED((NP, D), jnp.float32),
        [pltpu.SemaphoreType.DMA for _ in range(NB)],
        [pltpu.SemaphoreType.DMA for _ in range(NB)],
        [pltpu.SemaphoreType.DMA for _ in range(NB)],
    ],
)
def _spmm_kernel(x_hbm, edata_hbm, zeros_hbm, out_hbm,
                 ebufs, rbufs, cbufs, nbufs, rowss, acc_sh,
                 sems_e, sems_g, sems_s):
    cid, sid, wid = _worker_id()
    pltpu.sync_copy(zeros_hbm.at[pl.ds(sid * PPT, PPT)],
                    acc_sh.at[pl.ds(sid * PPT, PPT)])
    plsc.subcore_barrier()

    def scale_rows(rows, nbuf):
        for g in range(CH // 16):
            nv = lax.bitcast_convert_type(nbuf[pl.ds(g * 16, 16)],
                                          jnp.float32)
            for j in range(16):
                e = g * 16 + j
                spl = lax.gather(
                    nv, jnp.full((16, 1), j, jnp.int32),
                    lax.GatherDimensionNumbers(
                        offset_dims=(), collapsed_slice_dims=(0,),
                        start_index_map=(0,)),
                    slice_sizes=(1,),
                    mode=lax.GatherScatterMode.PROMISE_IN_BOUNDS)
                for s in range(D // 16):
                    sl = pl.ds(s * 16, 16)
                    rows[e, sl] = rows[e, sl] * spl

    def unpack_ebuf(b):
        for r, dst in ((0, rbufs[b]), (1, cbufs[b]), (2, nbufs[b])):
            for s in range(CH // 16):
                sl = pl.ds(s * 16, 16)
                dst[sl] = ebufs[b][r, sl]

    def start_scatter(b):
        return pltpu.async_copy(rowss[b], acc_sh.at[cbufs[b]], sems_s[b],
                                add=True)

    def drain_scatter(b):
        pltpu.make_async_copy(rowss[b], acc_sh.at[cbufs[b]],
                              sems_s[b]).wait()

    for b in range(NB):
        pltpu.async_copy(edata_hbm.at[b * NW + wid], ebufs[b], sems_e[b])

    @pl.loop(0, NOUT)
    def _outer(g):
        dg = []
        for b in range(NB):
            @pl.when(g > 0)
            def _drain():
                drain_scatter(b)
            pltpu.make_async_copy(edata_hbm.at[0], ebufs[b], sems_e[b]).wait()
            unpack_ebuf(b)
            dg.append(pltpu.async_copy(x_hbm.at[rbufs[b]], rowss[b],
                                       sems_g[b]))

            @pl.when(g < NOUT - 1)
            def _prefetch():
                pltpu.async_copy(
                    edata_hbm.at[((g + 1) * NB + b) * NW + wid],
                    ebufs[b], sems_e[b])
        for b in range(NB):
            dg[b].wait()
            scale_rows(rowss[b], nbufs[b])
            start_scatter(b)

    for b in range(NB):
        drain_scatter(b)

    @pl.when(wid < NEXTRA)
    def _extra():
        c = NFULLR * NW + wid
        pltpu.async_copy(edata_hbm.at[c], ebufs[0], sems_e[0]).wait()
        unpack_ebuf(0)
        pltpu.async_copy(x_hbm.at[rbufs[0]], rowss[0], sems_g[0]).wait()
        scale_rows(rowss[0], nbufs[0])
        start_scatter(0).wait()

    plsc.subcore_barrier()
    pltpu.sync_copy(acc_sh.at[pl.ds(sid * PPT, PPT)],
                    out_hbm.at[cid, pl.ds(sid * PPT, PPT)])


def _dinv_body(degp_ref, out_ref):
    s = degp_ref[0] + degp_ref[1]
    out_ref[...] = jnp.where(s > 0, lax.rsqrt(s), 0.0)


def _combine_body(p_ref, out_ref):
    out_ref[...] = p_ref[0] + p_ref[1]


def _layer_body(h_ref, s1_ref, p2_ref, w0_ref, w1_ref, w2_ref, b_ref, out_ref):
    w0 = w0_ref[...] - w2_ref[...]
    w2 = 2.0 * w2_ref[...]
    t2 = p2_ref[0] + p2_ref[1]
    acc = jnp.dot(h_ref[...], w0, preferred_element_type=jnp.float32)
    acc += jnp.dot(s1_ref[...], w1_ref[...], preferred_element_type=jnp.float32)
    acc += jnp.dot(t2, w2, preferred_element_type=jnp.float32)
    acc += b_ref[...]
    out_ref[...] = 1.0 / (1.0 + jnp.exp(-acc))


_RB = 1024

_combine = pl.pallas_call(
    _combine_body,
    grid=(NP // _RB,),
    in_specs=[pl.BlockSpec((NC, _RB, D), lambda i: (0, i, 0))],
    out_specs=pl.BlockSpec((_RB, D), lambda i: (i, 0)),
    out_shape=jax.ShapeDtypeStruct((NP, D), jnp.float32),
)

_layer = pl.pallas_call(
    _layer_body,
    grid=(NP // _RB,),
    in_specs=[
        pl.BlockSpec((_RB, D), lambda i: (i, 0)),
        pl.BlockSpec((_RB, D), lambda i: (i, 0)),
        pl.BlockSpec((NC, _RB, D), lambda i: (0, i, 0)),
        pl.BlockSpec((D, D), lambda i: (0, 0)),
        pl.BlockSpec((D, D), lambda i: (0, 0)),
        pl.BlockSpec((D, D), lambda i: (0, 0)),
        pl.BlockSpec((1, D), lambda i: (0, 0)),
    ],
    out_specs=pl.BlockSpec((_RB, D), lambda i: (i, 0)),
    out_shape=jax.ShapeDtypeStruct((NP, D), jnp.float32),
)

_dinv = pl.pallas_call(
    _dinv_body,
    out_shape=jax.ShapeDtypeStruct((NP // D, D), jnp.float32),
)


def kernel(x, edge_index, edge_weight, W1, b1, W2, b2, W3, b3):
    row = edge_index[0]
    col = edge_index[1]
    zeros_np = jnp.zeros((NP,), jnp.float32)
    zeros_nd = jnp.zeros((NP, D), jnp.float32)

    degp = _deg_kernel(row, edge_weight, zeros_np)
    dinv = _dinv(degp.reshape(NC, NP // D, D)).reshape(NP)
    edata = _norm_kernel(row, col, edge_weight, dinv)

    h = jnp.pad(x, ((0, NP - N), (0, 0)))
    for W, b in ((W1, b1), (W2, b2), (W3, b3)):
        p1 = _spmm_kernel(h, edata, zeros_nd)
        s1 = _combine(p1)
        p2 = _spmm_kernel(s1, edata, zeros_nd)
        h = _layer(h, s1, p2, W[0], W[1], W[2], b.reshape(1, D))
    return h[:N]
